# baseline (speedup 1.0000x reference)
import functools

import jax
import jax.numpy as jnp
import numpy as np
from jax import lax
from jax.experimental import pallas as pl
from jax.experimental.pallas import tpu as pltpu
from jax.experimental.pallas import tpu_sc as plsc

N_NODES = 10000
N_EDGES = 320000
D_FEAT = 128
D_EDGE = 16
STATE_DIM = 5
OUT_CH = 128

NC = 2
NS = 16
NW = NC * NS
CH = 40
NCH_PER_W = N_EDGES // (NW * CH)
NCH = NCH_PER_W

NBUF = 4
NIB = 8
NGRP = (NCH + NIB) // NIB

ROWS_MAIN = 624
ZROWS = 48

def _elu(v):
    return jnp.where(v > 0.0, v, jnp.exp(v) - 1.0)


def _node_proj_body(x_r, gs_r, wx_r, wg_r, wrx_r, wrg_r, bn_r, br_r, ei_r,
                    proj_r, root_r, src_r, dst_r):
    @pl.when(pl.program_id(0) == 0)
    def _():
        ei = ei_r[...]
        src_r[...] = ei[0]
        dst_r[...] = ei[1]
    xb = x_r[...]
    gb = gs_r[...]
    p = (jnp.dot(xb, wx_r[...], preferred_element_type=jnp.float32)
         + jnp.dot(gb, wg_r[...], preferred_element_type=jnp.float32)
         + bn_r[...])
    r = (jnp.dot(xb, wrx_r[...], preferred_element_type=jnp.float32)
         + jnp.dot(gb, wrg_r[...], preferred_element_type=jnp.float32)
         + br_r[...])
    proj_r[...] = p
    root_r[...] = _elu(r)


def _node_proj(x, gs8, wxT, wgT, wrxT, wrgT, bn, br, edge_index):
    bm = 1000
    grid = N_NODES // bm
    full = lambda i: (0, 0)
    return pl.pallas_call(
        _node_proj_body,
        grid=(grid,),
        in_specs=[
            pl.BlockSpec((bm, D_FEAT), lambda i: (i, 0)),
            pl.BlockSpec((bm, 8), lambda i: (i, 0)),
            pl.BlockSpec((D_FEAT, OUT_CH), full),
            pl.BlockSpec((8, OUT_CH), full),
            pl.BlockSpec((D_FEAT, OUT_CH), full),
            pl.BlockSpec((8, OUT_CH), full),
            pl.BlockSpec((1, OUT_CH), full),
            pl.BlockSpec((1, OUT_CH), full),
            pl.BlockSpec((2, N_EDGES), lambda i: (0, 0)),
        ],
        out_specs=[
            pl.BlockSpec((bm, OUT_CH), lambda i: (i, 0)),
            pl.BlockSpec((bm, OUT_CH), lambda i: (i, 0)),
            pl.BlockSpec((N_EDGES,), lambda i: (0,)),
            pl.BlockSpec((N_EDGES,), lambda i: (0,)),
        ],
        out_shape=[
            jax.ShapeDtypeStruct((N_NODES, OUT_CH), jnp.float32),
            jax.ShapeDtypeStruct((N_NODES, OUT_CH), jnp.float32),
            jax.ShapeDtypeStruct((N_EDGES,), jnp.int32),
            jax.ShapeDtypeStruct((N_EDGES,), jnp.int32),
        ],
    )(x, gs8, wxT, wgT, wrxT, wrgT, bn, br, edge_index)


def _eproj_body(eaT_r, we_r, out_r):
    out_r[...] = lax.dot_general(
        eaT_r[...], we_r[...],
        dimension_numbers=(((0,), (0,)), ((), ())),
        preferred_element_type=jnp.float32)


def _eproj(eaT, weT):
    bm = 12800
    grid = N_EDGES // bm
    return pl.pallas_call(
        _eproj_body,
        grid=(grid,),
        in_specs=[
            pl.BlockSpec((D_EDGE, bm), lambda i: (0, i)),
            pl.BlockSpec((D_EDGE, OUT_CH), lambda i: (0, 0)),
        ],
        out_specs=pl.BlockSpec((bm, OUT_CH), lambda i: (i, 0)),
        out_shape=jax.ShapeDtypeStruct((N_EDGES, OUT_CH), jnp.float32),
    )(eaT, weT)


def _sc_body(proj_hbm, eproj_hbm, dst_hbm, src_hbm, out_hbm,
             g3, e3, zbuf, acc,
             dix0, dix1, dix2, dix3, dix4, dix5, dix6, dix7,
             six0, six1, six2, six3, six4, six5, six6, six7,
             sid, sis, sg, se, ssc):
    c = lax.axis_index("c")
    s = lax.axis_index("s")
    wid = c * NS + s
    dixs = [dix0, dix1, dix2, dix3, dix4, dix5, dix6, dix7]
    sixs = [six0, six1, six2, six3, six4, six5, six6, six7]

    def issue_idx(jj, i):
        base = (wid * NCH + jj) * CH
        pltpu.async_copy(dst_hbm.at[pl.ds(base, CH)], dixs[i], sid.at[i])
        pltpu.async_copy(src_hbm.at[pl.ds(base, CH)], sixs[i], sis.at[i])

    def wait_dst_idx(i):
        pltpu.make_async_copy(dst_hbm.at[pl.ds(0, CH)], dixs[i],
                              sid.at[i]).wait()

    def wait_src_idx(i):
        pltpu.make_async_copy(src_hbm.at[pl.ds(0, CH)], sixs[i],
                              sis.at[i]).wait()

    def issue_fetch(jj, i, b):
        wait_dst_idx(i)
        pltpu.async_copy(proj_hbm.at[dixs[i]], g3.at[b], sg.at[b])
        base = (wid * NCH + jj) * CH
        pltpu.async_copy(eproj_hbm.at[pl.ds(base, CH)], e3.at[b], se.at[b])

    def wait_fetch(b):
        pltpu.make_async_copy(proj_hbm.at[pl.ds(0, CH)], g3.at[b],
                              sg.at[b]).wait()
        pltpu.make_async_copy(eproj_hbm.at[pl.ds(0, CH)], e3.at[b],
                              se.at[b]).wait()

    def wait_scat(b):
        pltpu.make_async_copy(proj_hbm.at[pl.ds(0, CH)], g3.at[b],
                              ssc.at[b]).wait()

    def compute(b):
        @plsc.parallel_loop(0, CH, step=1, unroll=2)
        def _(row):
            for k in range(D_FEAT // 16):
                v = (g3[b, row, pl.ds(16 * k, 16)]
                     + e3[b, row, pl.ds(16 * k, 16)])
                g3[b, row, pl.ds(16 * k, 16)] = jnp.where(
                    v > 0.0, v, jnp.exp(v) - 1.0)

    zero = jnp.zeros((16,), jnp.float32)

    def zrow(r, carry):
        for k in range(D_FEAT // 16):
            zbuf[r, pl.ds(k * 16, 16)] = zero
        return carry

    lax.fori_loop(0, ZROWS, zrow, 0)
    for t in range(ROWS_MAIN // ZROWS):
        pltpu.sync_copy(zbuf, acc.at[pl.ds(s * ROWS_MAIN + t * ZROWS, ZROWS)])

    @pl.when(s == NS - 1)
    def _():
        pltpu.sync_copy(zbuf.at[pl.ds(0, N_NODES - NS * ROWS_MAIN)],
                        acc.at[pl.ds(NS * ROWS_MAIN,
                                     N_NODES - NS * ROWS_MAIN)])

    issue_idx(0, 0)
    issue_idx(1, 1)
    issue_idx(2, 2)
    issue_fetch(0, 0, 0)
    issue_fetch(1, 1, 1)

    plsc.subcore_barrier()

    def group(g, carry):
        j0 = g * NIB
        for u in range(NIB):
            j = j0 + u
            b = u % NBUF
            b2 = (u + 2) % NBUF
            i2 = (u + 2) % NIB
            i3 = (u + 3) % NIB

            @pl.when(j + 3 < NCH)
            def _(j=j, i3=i3):
                issue_idx(j + 3, i3)

            @pl.when(j < NCH)
            def _(j=j, b=b):
                wait_fetch(b)
                compute(b)

            @pl.when(j + 2 < NCH)
            def _(j=j, b2=b2, i2=i2):
                @pl.when(j >= 2)
                def _():
                    wait_scat(b2)
                issue_fetch(j + 2, i2, b2)

            @pl.when(j < NCH)
            def _(j=j, b=b, u=u):
                wait_src_idx(u)
                pltpu.async_copy(g3.at[b], acc.at[sixs[u]], ssc.at[b],
                                 add=True)

        return carry

    lax.fori_loop(0, NGRP, group, 0)

    for t in range(NBUF):
        wait_scat((NCH - NBUF + t) % NBUF)

    plsc.subcore_barrier()
    for t in range(ROWS_MAIN // ZROWS):
        rs = s * ROWS_MAIN + t * ZROWS
        pltpu.sync_copy(acc.at[pl.ds(rs, ZROWS)],
                        out_hbm.at[c].at[pl.ds(rs, ZROWS)])

    @pl.when(s == NS - 1)
    def _():
        rs = NS * ROWS_MAIN
        pltpu.sync_copy(acc.at[pl.ds(rs, N_NODES - rs)],
                        out_hbm.at[c].at[pl.ds(rs, N_NODES - rs)])


def _sc_scatter(proj, eproj, dst, src):
    mesh = plsc.VectorSubcoreMesh(core_axis_name="c", subcore_axis_name="s",
                                  num_cores=NC, num_subcores=NS)
    f = functools.partial(
        pl.kernel,
        out_type=jax.ShapeDtypeStruct((NC, N_NODES, OUT_CH), jnp.float32),
        mesh=mesh,
        compiler_params=pltpu.CompilerParams(use_tc_tiling_on_sc=True),
        scratch_types=[
            pltpu.VMEM((NBUF, CH, OUT_CH), jnp.float32),
            pltpu.VMEM((NBUF, CH, OUT_CH), jnp.float32),
            pltpu.VMEM((ZROWS, OUT_CH), jnp.float32),
            pltpu.VMEM_SHARED((N_NODES, OUT_CH), jnp.float32),
        ] + [pltpu.VMEM((CH,), jnp.int32)] * 16 + [
            pltpu.SemaphoreType.DMA((NIB,)),
            pltpu.SemaphoreType.DMA((NIB,)),
            pltpu.SemaphoreType.DMA((NBUF,)),
            pltpu.SemaphoreType.DMA((NBUF,)),
            pltpu.SemaphoreType.DMA((NBUF,)),
        ],
    )(_sc_body)
    return f(proj, eproj, dst, src)


def _combine_body(root_r, p_r, out_r):
    out_r[...] = root_r[...] + p_r[0] + p_r[1]


def _combine(root, partials):
    bm = 1000
    grid = N_NODES // bm
    return pl.pallas_call(
        _combine_body,
        grid=(grid,),
        in_specs=[
            pl.BlockSpec((bm, OUT_CH), lambda i: (i, 0)),
            pl.BlockSpec((NC, bm, OUT_CH), lambda i: (0, i, 0)),
        ],
        out_specs=pl.BlockSpec((bm, OUT_CH), lambda i: (i, 0)),
        out_shape=jax.ShapeDtypeStruct((N_NODES, OUT_CH), jnp.float32),
    )(root, partials)


def kernel(x, edge_index, edge_attr, global_state, W_neg, b_neg,
           W_root, b_root):

    wxT = W_neg[:, :D_FEAT].T
    wgT = jnp.pad(W_neg[:, D_FEAT:D_FEAT + STATE_DIM], ((0, 0), (0, 3))).T
    weT = W_neg[:, D_FEAT + STATE_DIM:].T
    wrxT = W_root[:, :D_FEAT].T
    wrgT = jnp.pad(W_root[:, D_FEAT:], ((0, 0), (0, 3))).T
    gs8 = jnp.pad(global_state, ((0, 0), (0, 3)))

    proj, root, src, dst = _node_proj(
        x, gs8, wxT, wgT, wrxT, wrgT,
        b_neg.reshape(1, OUT_CH), b_root.reshape(1, OUT_CH),
        edge_index.astype(jnp.int32))
    eproj = _eproj(edge_attr.T, weT)
    partials = _sc_scatter(proj, eproj, dst, src)
    return _combine(root, partials)


# speedup vs baseline: 1.1825x; 1.1825x over previous
import functools

import jax
import jax.numpy as jnp
import numpy as np
from jax import lax
from jax.experimental import pallas as pl
from jax.experimental.pallas import tpu as pltpu
from jax.experimental.pallas import tpu_sc as plsc

N_NODES = 10000
N_EDGES = 320000
D_FEAT = 128
D_EDGE = 16
STATE_DIM = 5
OUT_CH = 128

NC = 2
NS = 16
NW = NC * NS
CH = 40
NCH_PER_W = N_EDGES // (NW * CH)
NCH = NCH_PER_W

NBUF = 4
NIB = 8
NGRP = (NCH + NIB) // NIB

ROWS_MAIN = 624
ZROWS = 48

def _elu(v):
    return jnp.where(v > 0.0, v, jnp.exp(v) - 1.0)


def _node_proj_body(x_r, gs_r, wx_r, wg_r, wrx_r, wrg_r, bn_r, br_r, ei_r,
                    proj_r, root_r, src_r, dst_r):
    @pl.when(pl.program_id(0) == 0)
    def _():
        ei = ei_r[...]
        src_r[...] = ei[0]
        dst_r[...] = ei[1]
    xb = x_r[...]
    gb = gs_r[...]
    p = (jnp.dot(xb, wx_r[...], preferred_element_type=jnp.float32)
         + jnp.dot(gb, wg_r[...], preferred_element_type=jnp.float32)
         + bn_r[...])
    r = (jnp.dot(xb, wrx_r[...], preferred_element_type=jnp.float32)
         + jnp.dot(gb, wrg_r[...], preferred_element_type=jnp.float32)
         + br_r[...])
    proj_r[...] = p
    root_r[...] = _elu(r)


def _node_proj(x, gs8, wxT, wgT, wrxT, wrgT, bn, br, edge_index):
    bm = 1000
    grid = N_NODES // bm
    full = lambda i: (0, 0)
    return pl.pallas_call(
        _node_proj_body,
        grid=(grid,),
        in_specs=[
            pl.BlockSpec((bm, D_FEAT), lambda i: (i, 0)),
            pl.BlockSpec((bm, 8), lambda i: (i, 0)),
            pl.BlockSpec((D_FEAT, OUT_CH), full),
            pl.BlockSpec((8, OUT_CH), full),
            pl.BlockSpec((D_FEAT, OUT_CH), full),
            pl.BlockSpec((8, OUT_CH), full),
            pl.BlockSpec((1, OUT_CH), full),
            pl.BlockSpec((1, OUT_CH), full),
            pl.BlockSpec((2, N_EDGES), lambda i: (0, 0)),
        ],
        out_specs=[
            pl.BlockSpec((bm, OUT_CH), lambda i: (i, 0)),
            pl.BlockSpec((bm, OUT_CH), lambda i: (i, 0)),
            pl.BlockSpec((N_EDGES,), lambda i: (0,)),
            pl.BlockSpec((N_EDGES,), lambda i: (0,)),
        ],
        out_shape=[
            jax.ShapeDtypeStruct((N_NODES, OUT_CH), jnp.float32),
            jax.ShapeDtypeStruct((N_NODES, OUT_CH), jnp.float32),
            jax.ShapeDtypeStruct((N_EDGES,), jnp.int32),
            jax.ShapeDtypeStruct((N_EDGES,), jnp.int32),
        ],
    )(x, gs8, wxT, wgT, wrxT, wrgT, bn, br, edge_index)


def _eproj_body(eaT_r, we_r, out_r):
    out_r[...] = lax.dot_general(
        eaT_r[...], we_r[...],
        dimension_numbers=(((0,), (0,)), ((), ())),
        preferred_element_type=jnp.float32)


def _eproj(eaT, weT):
    bm = 12800
    grid = N_EDGES // bm
    return pl.pallas_call(
        _eproj_body,
        grid=(grid,),
        in_specs=[
            pl.BlockSpec((D_EDGE, bm), lambda i: (0, i)),
            pl.BlockSpec((D_EDGE, OUT_CH), lambda i: (0, 0)),
        ],
        out_specs=pl.BlockSpec((bm, OUT_CH), lambda i: (i, 0)),
        out_shape=jax.ShapeDtypeStruct((N_EDGES, OUT_CH), jnp.float32),
    )(eaT, weT)


def _sc_body(proj_hbm, eproj_hbm, dst_hbm, src_hbm, out_hbm,
             g3, e3, zbuf, acc,
             dix0, dix1, dix2, dix3, dix4, dix5, dix6, dix7,
             six0, six1, six2, six3, six4, six5, six6, six7,
             sid, sis, sg, se, ssc):
    c = lax.axis_index("c")
    s = lax.axis_index("s")
    wid = c * NS + s
    dixs = [dix0, dix1, dix2, dix3, dix4, dix5, dix6, dix7]
    sixs = [six0, six1, six2, six3, six4, six5, six6, six7]

    def issue_idx(jj, i):
        base = (wid * NCH + jj) * CH
        pltpu.async_copy(dst_hbm.at[pl.ds(base, CH)], dixs[i], sid.at[i])
        pltpu.async_copy(src_hbm.at[pl.ds(base, CH)], sixs[i], sis.at[i])

    def wait_dst_idx(i):
        pltpu.make_async_copy(dst_hbm.at[pl.ds(0, CH)], dixs[i],
                              sid.at[i]).wait()

    def wait_src_idx(i):
        pltpu.make_async_copy(src_hbm.at[pl.ds(0, CH)], sixs[i],
                              sis.at[i]).wait()

    def issue_fetch(jj, i, b):
        wait_dst_idx(i)
        pltpu.async_copy(proj_hbm.at[dixs[i]], g3.at[b], sg.at[b])
        base = (wid * NCH + jj) * CH
        pltpu.async_copy(eproj_hbm.at[pl.ds(base, CH)], e3.at[b], se.at[b])

    def wait_fetch(b):
        pltpu.make_async_copy(proj_hbm.at[pl.ds(0, CH)], g3.at[b],
                              sg.at[b]).wait()
        pltpu.make_async_copy(eproj_hbm.at[pl.ds(0, CH)], e3.at[b],
                              se.at[b]).wait()

    def wait_scat(b):
        pltpu.make_async_copy(proj_hbm.at[pl.ds(0, CH)], g3.at[b],
                              ssc.at[b]).wait()

    def compute(b):
        def crow(r, inner):
            for u in range(2):
                row = 2 * r + u
                for k in range(D_FEAT // 16):
                    v = (g3[b, row, pl.ds(16 * k, 16)]
                         + e3[b, row, pl.ds(16 * k, 16)])
                    g3[b, row, pl.ds(16 * k, 16)] = jnp.where(
                        v > 0.0, v, jnp.exp(v) - 1.0)
            return inner

        lax.fori_loop(0, CH // 2, crow, 0)

    zero = jnp.zeros((16,), jnp.float32)

    def zrow(r, carry):
        for k in range(D_FEAT // 16):
            zbuf[r, pl.ds(k * 16, 16)] = zero
        return carry

    lax.fori_loop(0, ZROWS, zrow, 0)
    for t in range(ROWS_MAIN // ZROWS):
        pltpu.sync_copy(zbuf, acc.at[pl.ds(s * ROWS_MAIN + t * ZROWS, ZROWS)])

    @pl.when(s == NS - 1)
    def _():
        pltpu.sync_copy(zbuf.at[pl.ds(0, N_NODES - NS * ROWS_MAIN)],
                        acc.at[pl.ds(NS * ROWS_MAIN,
                                     N_NODES - NS * ROWS_MAIN)])

    issue_idx(0, 0)
    issue_idx(1, 1)
    issue_idx(2, 2)
    issue_fetch(0, 0, 0)
    issue_fetch(1, 1, 1)

    plsc.subcore_barrier()

    def group(g, carry):
        j0 = g * NIB
        for u in range(NIB):
            j = j0 + u
            b = u % NBUF
            b2 = (u + 2) % NBUF
            i2 = (u + 2) % NIB
            i3 = (u + 3) % NIB

            @pl.when(j + 3 < NCH)
            def _(j=j, i3=i3):
                issue_idx(j + 3, i3)

            @pl.when(j < NCH)
            def _(j=j, b=b):
                wait_fetch(b)
                compute(b)

            @pl.when(j + 2 < NCH)
            def _(j=j, b2=b2, i2=i2):
                @pl.when(j >= 2)
                def _():
                    wait_scat(b2)
                issue_fetch(j + 2, i2, b2)

            @pl.when(j < NCH)
            def _(j=j, b=b, u=u):
                wait_src_idx(u)
                pltpu.async_copy(g3.at[b], acc.at[sixs[u]], ssc.at[b],
                                 add=True)

        return carry

    lax.fori_loop(0, NGRP, group, 0)

    for t in range(NBUF):
        wait_scat((NCH - NBUF + t) % NBUF)

    plsc.subcore_barrier()
    for t in range(ROWS_MAIN // ZROWS):
        rs = s * ROWS_MAIN + t * ZROWS
        pltpu.sync_copy(acc.at[pl.ds(rs, ZROWS)],
                        out_hbm.at[c].at[pl.ds(rs, ZROWS)])

    @pl.when(s == NS - 1)
    def _():
        rs = NS * ROWS_MAIN
        pltpu.sync_copy(acc.at[pl.ds(rs, N_NODES - rs)],
                        out_hbm.at[c].at[pl.ds(rs, N_NODES - rs)])


def _sc_scatter(proj, eproj, dst, src):
    mesh = plsc.VectorSubcoreMesh(core_axis_name="c", subcore_axis_name="s",
                                  num_cores=NC, num_subcores=NS)
    f = functools.partial(
        pl.kernel,
        out_type=jax.ShapeDtypeStruct((NC, N_NODES, OUT_CH), jnp.float32),
        mesh=mesh,
        compiler_params=pltpu.CompilerParams(use_tc_tiling_on_sc=True),
        scratch_types=[
            pltpu.VMEM((NBUF, CH, OUT_CH), jnp.float32),
            pltpu.VMEM((NBUF, CH, OUT_CH), jnp.float32),
            pltpu.VMEM((ZROWS, OUT_CH), jnp.float32),
            pltpu.VMEM_SHARED((N_NODES, OUT_CH), jnp.float32),
        ] + [pltpu.VMEM((CH,), jnp.int32)] * 16 + [
            pltpu.SemaphoreType.DMA((NIB,)),
            pltpu.SemaphoreType.DMA((NIB,)),
            pltpu.SemaphoreType.DMA((NBUF,)),
            pltpu.SemaphoreType.DMA((NBUF,)),
            pltpu.SemaphoreType.DMA((NBUF,)),
        ],
    )(_sc_body)
    return f(proj, eproj, dst, src)


def _combine_body(root_r, p_r, out_r):
    out_r[...] = root_r[...] + p_r[0] + p_r[1]


def _combine(root, partials):
    bm = 1000
    grid = N_NODES // bm
    return pl.pallas_call(
        _combine_body,
        grid=(grid,),
        in_specs=[
            pl.BlockSpec((bm, OUT_CH), lambda i: (i, 0)),
            pl.BlockSpec((NC, bm, OUT_CH), lambda i: (0, i, 0)),
        ],
        out_specs=pl.BlockSpec((bm, OUT_CH), lambda i: (i, 0)),
        out_shape=jax.ShapeDtypeStruct((N_NODES, OUT_CH), jnp.float32),
    )(root, partials)


def kernel(x, edge_index, edge_attr, global_state, W_neg, b_neg,
           W_root, b_root):

    wxT = W_neg[:, :D_FEAT].T
    wgT = jnp.pad(W_neg[:, D_FEAT:D_FEAT + STATE_DIM], ((0, 0), (0, 3))).T
    weT = W_neg[:, D_FEAT + STATE_DIM:].T
    wrxT = W_root[:, :D_FEAT].T
    wrgT = jnp.pad(W_root[:, D_FEAT:], ((0, 0), (0, 3))).T
    gs8 = jnp.pad(global_state, ((0, 0), (0, 3)))

    proj, root, src, dst = _node_proj(
        x, gs8, wxT, wgT, wrxT, wrgT,
        b_neg.reshape(1, OUT_CH), b_root.reshape(1, OUT_CH),
        edge_index.astype(jnp.int32))
    eproj = _eproj(edge_attr.T, weT)
    partials = _sc_scatter(proj, eproj, dst, src)
    return _combine(root, partials)
